# baseline (speedup 1.0000x reference)
import jax
import jax.numpy as jnp
from jax import lax
from jax.experimental import pallas as pl
from jax.experimental.pallas import tpu as pltpu
from jax.experimental.pallas import tpu_sc as plsc

_B, _S, _D = 4, 8192, 1024
_BS = 512

_NC, _NS, _L = 2, 16, 16
_NW = _NC * _NS
_SC_ROWS = 512
_S0 = _S - _SC_ROWS
_RW = _SC_ROWS // _NW
_CHUNK = _RW * _D


def _add_kernel(x_ref, pos_ref, out_ref):
    out_ref[...] = x_ref[...] + pos_ref[...]


def _tc_call(x, pos_emb):
    b, seq_len, dim = x.shape
    return pl.pallas_call(
        _add_kernel,
        grid=(seq_len // _BS,),
        in_specs=[
            pl.BlockSpec((b, _BS, dim), lambda s: (0, s, 0)),
            pl.BlockSpec((_BS, dim), lambda s: (s, 0)),
        ],
        out_specs=pl.BlockSpec((b, _BS, dim), lambda s: (0, s, 0)),
        out_shape=jax.ShapeDtypeStruct(x.shape, x.dtype),
        compiler_params=pltpu.CompilerParams(
            dimension_semantics=("parallel",),
        ),
    )(x, pos_emb)


def _sc_body(x_hbm, pos_hbm, out_hbm, pv, x0, x1, s0, s1, osem):
    wid = lax.axis_index("s") * _NC + lax.axis_index("c")
    row0 = _S0 + wid * _RW
    xbuf = (x0, x1)
    isem = (s0, s1)

    def x_off(b):
        return b * (_S * _D) + row0 * _D

    def o_off(b):
        return b * (_SC_ROWS * _D) + wid * _RW * _D

    def start_load(b):
        return pltpu.async_copy(
            x_hbm.at[pl.ds(x_off(b), _CHUNK)], xbuf[b % 2], isem[b % 2])

    in_d, out_d = {}, {}
    in_d[0] = start_load(0)
    pltpu.sync_copy(pos_hbm.at[pl.ds(row0 * _D, _CHUNK)], pv)
    for b in range(_B):
        if b >= 1:
            out_d[b - 1].wait()
        if b + 1 < _B:
            in_d[b + 1] = start_load(b + 1)
        in_d[b].wait()
        xv = xbuf[b % 2]

        @plsc.parallel_loop(0, _CHUNK, step=_L, unroll=8)
        def vec_body(i):
            sl = pl.ds(i, _L)
            xv[sl] = xv[sl] + pv[sl]

        out_d[b] = pltpu.async_copy(
            xv, out_hbm.at[pl.ds(o_off(b), _CHUNK)], osem)
    out_d[_B - 1].wait()


def _sc_call(x, pos_emb):
    b, seq_len, dim = x.shape
    mesh = plsc.VectorSubcoreMesh(core_axis_name="c", subcore_axis_name="s")
    out = pl.kernel(
        _sc_body,
        out_type=jax.ShapeDtypeStruct((b * _SC_ROWS * dim,), x.dtype),
        mesh=mesh,
        scratch_types=[
            pltpu.VMEM((_CHUNK,), jnp.float32),
            pltpu.VMEM((_CHUNK,), jnp.float32),
            pltpu.VMEM((_CHUNK,), jnp.float32),
            pltpu.SemaphoreType.DMA,
            pltpu.SemaphoreType.DMA,
            pltpu.SemaphoreType.DMA,
        ],
    )(x.reshape(b * seq_len * dim), pos_emb.reshape(seq_len * dim))
    return out.reshape(b, _SC_ROWS, dim)


def kernel(x, pos_emb):
    tc_out = _tc_call(x, pos_emb)
    sc_out = _sc_call(x, pos_emb)
    return lax.dynamic_update_slice(tc_out, sc_out, (0, _S0, 0))


# speedup vs baseline: 1.7292x; 1.7292x over previous
import jax
import jax.numpy as jnp
from jax import lax
from jax.experimental import pallas as pl
from jax.experimental.pallas import tpu as pltpu
from jax.experimental.pallas import tpu_sc as plsc

_B, _S, _D = 4, 8192, 1024
_NC, _NS, _L = 2, 16, 16
_NW = _NC * _NS
_ROWS_W = _S // _NW
_R = 16
_NCHUNK = _ROWS_W // _R
_T = _NCHUNK * _B
_NBUF = 4


def _sc_body(x_hbm, pos_hbm, out_hbm,
             p0, p1, x0, x1, x2, x3,
             psem, s0, s1, s2, s3, osem):
    wid = lax.axis_index("s") * _NC + lax.axis_index("c")
    row0 = wid * _ROWS_W
    xbuf = (x0, x1, x2, x3)
    isem = (s0, s1, s2, s3)
    pbuf = (p0, p1)

    def rows(t):
        return pl.ds(row0 + (t // _B) * _R, _R)

    def start_load(t):
        return pltpu.async_copy(
            x_hbm.at[t % _B, rows(t), :], xbuf[t % _NBUF], isem[t % _NBUF])

    def start_pos(c):
        return pltpu.async_copy(
            pos_hbm.at[pl.ds(row0 + c * _R, _R), :], pbuf[c % 2], psem)

    in_d, out_d, pos_d = {}, {}, {}
    pos_d[0] = start_pos(0)
    for t in range(min(_NBUF - 1, _T)):
        in_d[t] = start_load(t)

    for t in range(_T):
        c, b = t // _B, t % _B
        if t >= 1:
            out_d[t - 1].wait()
        if t + _NBUF - 1 < _T:
            in_d[t + _NBUF - 1] = start_load(t + _NBUF - 1)
        if b == 0:
            pos_d[c].wait()
            if c + 1 < _NCHUNK:
                pos_d[c + 1] = start_pos(c + 1)
        in_d[t].wait()
        xv = xbuf[t % _NBUF]
        pv = pbuf[c % 2]

        @plsc.parallel_loop(0, _R * _D, step=_L, unroll=8)
        def vec_body(i):
            r = i // _D
            sl = pl.ds(i - r * _D, _L)
            xv[r, sl] = xv[r, sl] + pv[r, sl]

        out_d[t] = pltpu.async_copy(xv, out_hbm.at[b, rows(t), :], osem)
    out_d[_T - 1].wait()


def kernel(x, pos_emb):
    b, seq_len, dim = x.shape
    mesh = plsc.VectorSubcoreMesh(core_axis_name="c", subcore_axis_name="s")
    return pl.kernel(
        _sc_body,
        out_type=jax.ShapeDtypeStruct((b, seq_len, dim), x.dtype),
        mesh=mesh,
        scratch_types=[
            pltpu.VMEM((_R, _D), jnp.float32),
            pltpu.VMEM((_R, _D), jnp.float32),
            pltpu.VMEM((_R, _D), jnp.float32),
            pltpu.VMEM((_R, _D), jnp.float32),
            pltpu.VMEM((_R, _D), jnp.float32),
            pltpu.VMEM((_R, _D), jnp.float32),
            pltpu.SemaphoreType.DMA,
            pltpu.SemaphoreType.DMA,
            pltpu.SemaphoreType.DMA,
            pltpu.SemaphoreType.DMA,
            pltpu.SemaphoreType.DMA,
            pltpu.SemaphoreType.DMA,
        ],
    )(x, pos_emb)


# speedup vs baseline: 1.7688x; 1.0229x over previous
import jax
import jax.numpy as jnp
from jax import lax
from jax.experimental import pallas as pl
from jax.experimental.pallas import tpu as pltpu
from jax.experimental.pallas import tpu_sc as plsc

_B, _S, _D = 4, 8192, 1024
_BS = 512
_SC_ROWS = 3072
_S0 = _S - _SC_ROWS

_NC, _NS, _L = 2, 16, 16
_NW = _NC * _NS
_ROWS_W = _SC_ROWS // _NW
_R = 16
_NCHUNK = _ROWS_W // _R
_T = _NCHUNK * _B
_NBUF = 4


def _add_kernel(x_ref, pos_ref, out_ref):
    out_ref[...] = x_ref[...] + pos_ref[...]


def _tc_call(x, pos_emb):
    b, seq_len, dim = x.shape
    return pl.pallas_call(
        _add_kernel,
        grid=(_S0 // _BS,),
        in_specs=[
            pl.BlockSpec((b, _BS, dim), lambda s: (0, s, 0)),
            pl.BlockSpec((_BS, dim), lambda s: (s, 0)),
        ],
        out_specs=pl.BlockSpec((b, _BS, dim), lambda s: (0, s, 0)),
        out_shape=jax.ShapeDtypeStruct(x.shape, x.dtype),
        compiler_params=pltpu.CompilerParams(
            dimension_semantics=("parallel",),
        ),
    )(x, pos_emb)


def _sc_body(x_hbm, pos_hbm, out_hbm,
             p0, p1, x0, x1, x2, x3,
             psem, s0, s1, s2, s3, osem):
    wid = lax.axis_index("s") * _NC + lax.axis_index("c")
    row0 = _S0 + wid * _ROWS_W
    orow0 = wid * _ROWS_W
    xbuf = (x0, x1, x2, x3)
    isem = (s0, s1, s2, s3)
    pbuf = (p0, p1)

    def start_load(t):
        sl = pl.ds(row0 + (t // _B) * _R, _R)
        return pltpu.async_copy(
            x_hbm.at[t % _B, sl, :], xbuf[t % _NBUF], isem[t % _NBUF])

    def start_pos(c):
        return pltpu.async_copy(
            pos_hbm.at[pl.ds(row0 + c * _R, _R), :], pbuf[c % 2], psem)

    in_d, out_d, pos_d = {}, {}, {}
    pos_d[0] = start_pos(0)
    for t in range(min(_NBUF - 1, _T)):
        in_d[t] = start_load(t)

    for t in range(_T):
        c, b = t // _B, t % _B
        if t >= 1:
            out_d[t - 1].wait()
        if t + _NBUF - 1 < _T:
            in_d[t + _NBUF - 1] = start_load(t + _NBUF - 1)
        if b == 0:
            pos_d[c].wait()
            if c + 1 < _NCHUNK:
                pos_d[c + 1] = start_pos(c + 1)
        in_d[t].wait()
        xv = xbuf[t % _NBUF]
        pv = pbuf[c % 2]

        @plsc.parallel_loop(0, _R * _D, step=_L, unroll=8)
        def vec_body(i):
            r = i // _D
            sl = pl.ds(i - r * _D, _L)
            xv[r, sl] = xv[r, sl] + pv[r, sl]

        out_d[t] = pltpu.async_copy(
            xv, out_hbm.at[b, pl.ds(orow0 + c * _R, _R), :], osem)
    out_d[_T - 1].wait()


def _sc_call(x, pos_emb):
    b, seq_len, dim = x.shape
    mesh = plsc.VectorSubcoreMesh(core_axis_name="c", subcore_axis_name="s")
    return pl.kernel(
        _sc_body,
        out_type=jax.ShapeDtypeStruct((b, _SC_ROWS, dim), x.dtype),
        mesh=mesh,
        scratch_types=[
            pltpu.VMEM((_R, _D), jnp.float32),
            pltpu.VMEM((_R, _D), jnp.float32),
            pltpu.VMEM((_R, _D), jnp.float32),
            pltpu.VMEM((_R, _D), jnp.float32),
            pltpu.VMEM((_R, _D), jnp.float32),
            pltpu.VMEM((_R, _D), jnp.float32),
            pltpu.SemaphoreType.DMA,
            pltpu.SemaphoreType.DMA,
            pltpu.SemaphoreType.DMA,
            pltpu.SemaphoreType.DMA,
            pltpu.SemaphoreType.DMA,
            pltpu.SemaphoreType.DMA,
        ],
    )(x, pos_emb)


def kernel(x, pos_emb):
    sc_out = _sc_call(x, pos_emb)
    tc_out = _tc_call(x, pos_emb)
    return lax.dynamic_update_slice(tc_out, sc_out, (0, _S0, 0))


# speedup vs baseline: 2.7497x; 1.5545x over previous
import jax
import jax.numpy as jnp
from jax.experimental import pallas as pl
from jax.experimental.pallas import tpu as pltpu

_BS = 512


def _add_kernel(x_ref, pos_ref, out_ref):
    out_ref[...] = x_ref[...] + pos_ref[...]


def kernel(x, pos_emb):
    b, seq_len, dim = x.shape
    return pl.pallas_call(
        _add_kernel,
        grid=(seq_len // _BS,),
        in_specs=[
            pl.BlockSpec((b, _BS, dim), lambda s: (0, s, 0)),
            pl.BlockSpec((_BS, dim), lambda s: (s, 0)),
        ],
        out_specs=pl.BlockSpec((b, _BS, dim), lambda s: (0, s, 0)),
        out_shape=jax.ShapeDtypeStruct(x.shape, x.dtype),
        compiler_params=pltpu.CompilerParams(
            dimension_semantics=("parallel",),
        ),
    )(x, pos_emb)
